# Initial kernel scaffold; baseline (speedup 1.0000x reference)
import jax
import jax.numpy as jnp
from jax.experimental import pallas as pl


def kernel(x, stem_conv1_w, b0_bn1_scale, b0_bn1_shift, b0_conv1_w, b0_bn2_scale, b0_bn2_shift, b0_conv2_w, b0_shortcut_w, b1_bn1_scale, b1_bn1_shift, b1_conv1_w, b1_bn2_scale, b1_bn2_shift, b1_conv2_w, b2_bn1_scale, b2_bn1_shift, b2_conv1_w, b2_bn2_scale, b2_bn2_shift, b2_conv2_w, b3_bn1_scale, b3_bn1_shift, b3_conv1_w, b3_bn2_scale, b3_bn2_shift, b3_conv2_w, b4_bn1_scale, b4_bn1_shift, b4_conv1_w, b4_bn2_scale, b4_bn2_shift, b4_conv2_w, b4_shortcut_w, b5_bn1_scale, b5_bn1_shift, b5_conv1_w, b5_bn2_scale, b5_bn2_shift, b5_conv2_w, b6_bn1_scale, b6_bn1_shift, b6_conv1_w, b6_bn2_scale, b6_bn2_shift, b6_conv2_w, b7_bn1_scale, b7_bn1_shift, b7_conv1_w, b7_bn2_scale, b7_bn2_shift, b7_conv2_w, b8_bn1_scale, b8_bn1_shift, b8_conv1_w, b8_bn2_scale, b8_bn2_shift, b8_conv2_w, b8_shortcut_w, b9_bn1_scale, b9_bn1_shift, b9_conv1_w, b9_bn2_scale, b9_bn2_shift, b9_conv2_w, b10_bn1_scale, b10_bn1_shift, b10_conv1_w, b10_bn2_scale, b10_bn2_shift, b10_conv2_w, b11_bn1_scale, b11_bn1_shift, b11_conv1_w, b11_bn2_scale, b11_bn2_shift, b11_conv2_w, bn_final_scale, bn_final_shift):
    raise NotImplementedError("write your pallas kernel here")


# speedup vs baseline: 3.3698x; 3.3698x over previous
import jax
import jax.numpy as jnp
from jax.experimental import pallas as pl
from jax.experimental.pallas import tpu as pltpu

BF = jnp.bfloat16


def _zero_border(ref, b, hp, wp, c, dtype):
    zr = jnp.zeros((b, 1, wp, c), dtype)
    ref[:, 0:1, :, :] = zr
    ref[:, hp - 1:hp, :, :] = zr
    zc = jnp.zeros((b, hp, 1, c), dtype)
    ref[:, :, 0:1, :] = zc
    ref[:, :, wp - 1:wp, :] = zc


def _conv9(pad_ref, w_ref, b, ho, wo, cin, cout):
    m = b * ho * wo
    acc = jnp.zeros((m, cout), jnp.float32)
    for ki in range(3):
        for kj in range(3):
            win = pad_ref[:, ki:ki + ho, kj:kj + wo, :]
            acc = acc + jnp.dot(win.reshape(m, cin), w_ref[ki, kj],
                                preferred_element_type=jnp.float32)
    return acc


def _decimate2(v, b, ho, wo, c):
    v = v.reshape(b, ho, 2, 2 * wo, c)[:, :, 0]
    return v.reshape(b, ho, wo, 2, c)[:, :, :, 0]


def _conv9_s2(pad_ref, w_ref, b, ho, wo, cin, cout):
    m = b * ho * wo
    acc = jnp.zeros((m, cout), jnp.float32)
    for ki in range(3):
        for kj in range(3):
            slab = pad_ref[:, ki:ki + 2 * ho, kj:kj + 2 * wo, :]
            win = _decimate2(slab, b, ho, wo, cin)
            acc = acc + jnp.dot(win.reshape(m, cin), w_ref[ki, kj],
                                preferred_element_type=jnp.float32)
    return acc


def _make_block_body(b, h, w, cin, cout, stride, proj, fin):
    ho, wo = h // stride, w // stride
    m = b * ho * wo

    def body(*refs):
        it = iter(refs)
        x_ref = next(it)
        s1_ref = next(it)
        t1_ref = next(it)
        w1_ref = next(it)
        s2_ref = next(it)
        t2_ref = next(it)
        w2_ref = next(it)
        sw_ref = next(it) if proj else None
        if fin:
            fs_ref = next(it)
            ft_ref = next(it)
        o_ref = next(it)
        pad1 = next(it)
        pad2 = next(it)

        x = x_ref[...]
        a1 = jnp.maximum(x * s1_ref[...] + t1_ref[...], 0.0).astype(BF)
        _zero_border(pad1, b, h + 2, w + 2, cin, BF)
        pad1[:, 1:h + 1, 1:w + 1, :] = a1

        if stride == 1:
            acc1 = _conv9(pad1, w1_ref, b, ho, wo, cin, cout)
        else:
            acc1 = _conv9_s2(pad1, w1_ref, b, ho, wo, cin, cout)

        a2 = jnp.maximum(acc1 * s2_ref[...] + t2_ref[...], 0.0)
        _zero_border(pad2, b, ho + 2, wo + 2, cout, BF)
        pad2[:, 1:ho + 1, 1:wo + 1, :] = a2.astype(BF).reshape(b, ho, wo, cout)

        acc2 = _conv9(pad2, w2_ref, b, ho, wo, cout, cout)

        if proj:
            xs = _decimate2(x, b, ho, wo, cin) if stride == 2 else x
            acc2 = acc2 + jnp.dot(xs.reshape(m, cin).astype(BF), sw_ref[...],
                                  preferred_element_type=jnp.float32)
        else:
            acc2 = acc2 + x.reshape(m, cout)

        if fin:
            acc2 = jnp.maximum(acc2 * fs_ref[...] + ft_ref[...], 0.0)

        o_ref[...] = acc2.reshape(b, ho, wo, cout)

    return body


def _res_block(x, s1, t1, w1, s2, t2, w2, sw, stride, bb, post_bn):
    n, h, w, cin = x.shape
    cout = w1.shape[-1]
    ho, wo = h // stride, w // stride
    proj = sw is not None
    fin = post_bn is not None

    inputs = [x, s1.reshape(1, cin), t1.reshape(1, cin), w1.astype(BF),
              s2.reshape(1, cout), t2.reshape(1, cout), w2.astype(BF)]
    in_specs = [
        pl.BlockSpec((bb, h, w, cin), lambda i: (i, 0, 0, 0)),
        pl.BlockSpec((1, cin), lambda i: (0, 0)),
        pl.BlockSpec((1, cin), lambda i: (0, 0)),
        pl.BlockSpec((3, 3, cin, cout), lambda i: (0, 0, 0, 0)),
        pl.BlockSpec((1, cout), lambda i: (0, 0)),
        pl.BlockSpec((1, cout), lambda i: (0, 0)),
        pl.BlockSpec((3, 3, cout, cout), lambda i: (0, 0, 0, 0)),
    ]
    if proj:
        inputs.append(sw.astype(BF))
        in_specs.append(pl.BlockSpec((cin, cout), lambda i: (0, 0)))
    if fin:
        fs, ft = post_bn
        inputs += [fs.reshape(1, cout), ft.reshape(1, cout)]
        in_specs += [pl.BlockSpec((1, cout), lambda i: (0, 0)),
                     pl.BlockSpec((1, cout), lambda i: (0, 0))]

    flops = 2 * n * ho * wo * 9 * cout * (cin + cout)
    if proj:
        flops += 2 * n * ho * wo * cin * cout
    bytes_accessed = (n * h * w * cin + n * ho * wo * cout) * 4 + sum(
        int(a.size) * a.dtype.itemsize for a in inputs[1:])

    return pl.pallas_call(
        _make_block_body(bb, h, w, cin, cout, stride, proj, fin),
        out_shape=jax.ShapeDtypeStruct((n, ho, wo, cout), jnp.float32),
        grid=(n // bb,),
        in_specs=in_specs,
        out_specs=pl.BlockSpec((bb, ho, wo, cout), lambda i: (i, 0, 0, 0)),
        scratch_shapes=[pltpu.VMEM((bb, h + 2, w + 2, cin), BF),
                        pltpu.VMEM((bb, ho + 2, wo + 2, cout), BF)],
        compiler_params=pltpu.CompilerParams(
            dimension_semantics=("parallel",)),
        cost_estimate=pl.CostEstimate(flops=flops, transcendentals=0,
                                      bytes_accessed=bytes_accessed),
    )(*inputs)


def _make_stem_body(b, h, w, cin, cout):
    def body(x_ref, w_ref, o_ref, pad):
        _zero_border(pad, b, h + 2, w + 2, cin, jnp.float32)
        pad[:, 1:h + 1, 1:w + 1, :] = x_ref[...]
        acc = _conv9(pad, w_ref, b, h, w, cin, cout)
        o_ref[...] = acc.reshape(b, h, w, cout)
    return body


def _stem(x, w, bb):
    n, h, wd, cin = x.shape
    cout = w.shape[-1]
    return pl.pallas_call(
        _make_stem_body(bb, h, wd, cin, cout),
        out_shape=jax.ShapeDtypeStruct((n, h, wd, cout), jnp.float32),
        grid=(n // bb,),
        in_specs=[pl.BlockSpec((bb, h, wd, cin), lambda i: (i, 0, 0, 0)),
                  pl.BlockSpec((3, 3, cin, cout), lambda i: (0, 0, 0, 0))],
        out_specs=pl.BlockSpec((bb, h, wd, cout), lambda i: (i, 0, 0, 0)),
        scratch_shapes=[pltpu.VMEM((bb, h + 2, wd + 2, cin), jnp.float32)],
        compiler_params=pltpu.CompilerParams(
            dimension_semantics=("parallel",)),
        cost_estimate=pl.CostEstimate(
            flops=2 * n * h * wd * 9 * cin * cout, transcendentals=0,
            bytes_accessed=(n * h * wd * (cin + cout)) * 4),
    )(x, w)


def kernel(x, stem_conv1_w, b0_bn1_scale, b0_bn1_shift, b0_conv1_w, b0_bn2_scale, b0_bn2_shift, b0_conv2_w, b0_shortcut_w, b1_bn1_scale, b1_bn1_shift, b1_conv1_w, b1_bn2_scale, b1_bn2_shift, b1_conv2_w, b2_bn1_scale, b2_bn1_shift, b2_conv1_w, b2_bn2_scale, b2_bn2_shift, b2_conv2_w, b3_bn1_scale, b3_bn1_shift, b3_conv1_w, b3_bn2_scale, b3_bn2_shift, b3_conv2_w, b4_bn1_scale, b4_bn1_shift, b4_conv1_w, b4_bn2_scale, b4_bn2_shift, b4_conv2_w, b4_shortcut_w, b5_bn1_scale, b5_bn1_shift, b5_conv1_w, b5_bn2_scale, b5_bn2_shift, b5_conv2_w, b6_bn1_scale, b6_bn1_shift, b6_conv1_w, b6_bn2_scale, b6_bn2_shift, b6_conv2_w, b7_bn1_scale, b7_bn1_shift, b7_conv1_w, b7_bn2_scale, b7_bn2_shift, b7_conv2_w, b8_bn1_scale, b8_bn1_shift, b8_conv1_w, b8_bn2_scale, b8_bn2_shift, b8_conv2_w, b8_shortcut_w, b9_bn1_scale, b9_bn1_shift, b9_conv1_w, b9_bn2_scale, b9_bn2_shift, b9_conv2_w, b10_bn1_scale, b10_bn1_shift, b10_conv1_w, b10_bn2_scale, b10_bn2_shift, b10_conv2_w, b11_bn1_scale, b11_bn1_shift, b11_conv1_w, b11_bn2_scale, b11_bn2_shift, b11_conv2_w, bn_final_scale, bn_final_shift):
    blocks = [
        (b0_bn1_scale, b0_bn1_shift, b0_conv1_w, b0_bn2_scale, b0_bn2_shift, b0_conv2_w, b0_shortcut_w, 1, 4),
        (b1_bn1_scale, b1_bn1_shift, b1_conv1_w, b1_bn2_scale, b1_bn2_shift, b1_conv2_w, None, 1, 4),
        (b2_bn1_scale, b2_bn1_shift, b2_conv1_w, b2_bn2_scale, b2_bn2_shift, b2_conv2_w, None, 1, 4),
        (b3_bn1_scale, b3_bn1_shift, b3_conv1_w, b3_bn2_scale, b3_bn2_shift, b3_conv2_w, None, 1, 4),
        (b4_bn1_scale, b4_bn1_shift, b4_conv1_w, b4_bn2_scale, b4_bn2_shift, b4_conv2_w, b4_shortcut_w, 2, 8),
        (b5_bn1_scale, b5_bn1_shift, b5_conv1_w, b5_bn2_scale, b5_bn2_shift, b5_conv2_w, None, 1, 8),
        (b6_bn1_scale, b6_bn1_shift, b6_conv1_w, b6_bn2_scale, b6_bn2_shift, b6_conv2_w, None, 1, 8),
        (b7_bn1_scale, b7_bn1_shift, b7_conv1_w, b7_bn2_scale, b7_bn2_shift, b7_conv2_w, None, 1, 8),
        (b8_bn1_scale, b8_bn1_shift, b8_conv1_w, b8_bn2_scale, b8_bn2_shift, b8_conv2_w, b8_shortcut_w, 2, 8),
        (b9_bn1_scale, b9_bn1_shift, b9_conv1_w, b9_bn2_scale, b9_bn2_shift, b9_conv2_w, None, 1, 8),
        (b10_bn1_scale, b10_bn1_shift, b10_conv1_w, b10_bn2_scale, b10_bn2_shift, b10_conv2_w, None, 1, 8),
        (b11_bn1_scale, b11_bn1_shift, b11_conv1_w, b11_bn2_scale, b11_bn2_shift, b11_conv2_w, None, 1, 8),
    ]
    h = jnp.transpose(x, (0, 2, 3, 1))
    h = _stem(h, stem_conv1_w, 8)
    last = len(blocks) - 1
    for i, (s1, t1, w1, s2, t2, w2, sw, stride, bb) in enumerate(blocks):
        post_bn = (bn_final_scale, bn_final_shift) if i == last else None
        h = _res_block(h, s1, t1, w1, s2, t2, w2, sw, stride, bb, post_bn)
    return jnp.transpose(h, (0, 3, 1, 2))


# speedup vs baseline: 3.3893x; 1.0058x over previous
import jax
import jax.numpy as jnp
from jax.experimental import pallas as pl
from jax.experimental.pallas import tpu as pltpu

BF = jnp.bfloat16


def _zero_border(ref, b, hp, wp, c, dtype):
    zr = jnp.zeros((b, 1, wp, c), dtype)
    ref[:, 0:1, :, :] = zr
    ref[:, hp - 1:hp, :, :] = zr
    zc = jnp.zeros((b, hp, 1, c), dtype)
    ref[:, :, 0:1, :] = zc
    ref[:, :, wp - 1:wp, :] = zc


def _cpad(c):
    return -(-c // 128) * 128


def _conv9(pad_ref, w_ref, pat_ref, b, ho, wo, cin, cout):
    m = b * ho * wo
    cp = _cpad(cin)
    kk = 9 * cp
    if cp > cin:
        zpad = jnp.zeros((m, cp - cin), BF)
        for t in range(9):
            pat_ref[:, t * cp + cin:(t + 1) * cp] = zpad
    t = 0
    for ki in range(3):
        for kj in range(3):
            win = pad_ref[:, ki:ki + ho, kj:kj + wo, :]
            pat_ref[:, t * cp:t * cp + cin] = win.reshape(m, cin)
            t += 1
    return jnp.dot(pat_ref[:, 0:kk], w_ref[...],
                   preferred_element_type=jnp.float32)


def _decimate2(v, b, ho, wo, c):
    v = v.reshape(b, ho, 2, 2 * wo, c)[:, :, 0]
    return v.reshape(b, ho, wo, 2, c)[:, :, :, 0]


def _conv9_s2(pad_ref, w_ref, pat_ref, b, ho, wo, cin, cout):
    m = b * ho * wo
    cp = _cpad(cin)
    kk = 9 * cp
    if cp > cin:
        zpad = jnp.zeros((m, cp - cin), BF)
        for t in range(9):
            pat_ref[:, t * cp + cin:(t + 1) * cp] = zpad
    t = 0
    for ki in range(3):
        for kj in range(3):
            slab = pad_ref[:, ki:ki + 2 * ho, kj:kj + 2 * wo, :]
            win = _decimate2(slab, b, ho, wo, cin)
            pat_ref[:, t * cp:t * cp + cin] = win.reshape(m, cin)
            t += 1
    return jnp.dot(pat_ref[:, 0:kk], w_ref[...],
                   preferred_element_type=jnp.float32)


def _prep_w(w, cin, cout):
    cp = _cpad(cin)
    wp = jnp.pad(w.astype(BF), ((0, 0), (0, 0), (0, cp - cin), (0, 0)))
    return wp.reshape(9 * cp, cout)


def _make_block_body(b, h, w, cin, cout, stride, proj, fin):
    ho, wo = h // stride, w // stride
    m = b * ho * wo

    def body(*refs):
        it = iter(refs)
        x_ref = next(it)
        s1_ref = next(it)
        t1_ref = next(it)
        w1_ref = next(it)
        s2_ref = next(it)
        t2_ref = next(it)
        w2_ref = next(it)
        sw_ref = next(it) if proj else None
        if fin:
            fs_ref = next(it)
            ft_ref = next(it)
        o_ref = next(it)
        pad1 = next(it)
        pad2 = next(it)
        pat = next(it)

        x = x_ref[...]
        a1 = jnp.maximum(x * s1_ref[...] + t1_ref[...], 0.0).astype(BF)
        _zero_border(pad1, b, h + 2, w + 2, cin, BF)
        pad1[:, 1:h + 1, 1:w + 1, :] = a1

        if stride == 1:
            acc1 = _conv9(pad1, w1_ref, pat, b, ho, wo, cin, cout)
        else:
            acc1 = _conv9_s2(pad1, w1_ref, pat, b, ho, wo, cin, cout)

        a2 = jnp.maximum(acc1 * s2_ref[...] + t2_ref[...], 0.0)
        _zero_border(pad2, b, ho + 2, wo + 2, cout, BF)
        pad2[:, 1:ho + 1, 1:wo + 1, :] = a2.astype(BF).reshape(b, ho, wo, cout)

        acc2 = _conv9(pad2, w2_ref, pat, b, ho, wo, cout, cout)

        if proj:
            xs = _decimate2(x, b, ho, wo, cin) if stride == 2 else x
            acc2 = acc2 + jnp.dot(xs.reshape(m, cin).astype(BF), sw_ref[...],
                                  preferred_element_type=jnp.float32)
        else:
            acc2 = acc2 + x.reshape(m, cout)

        if fin:
            acc2 = jnp.maximum(acc2 * fs_ref[...] + ft_ref[...], 0.0)

        o_ref[...] = acc2.reshape(b, ho, wo, cout)

    return body


def _res_block(x, s1, t1, w1, s2, t2, w2, sw, stride, bb, post_bn):
    n, h, w, cin = x.shape
    cout = w1.shape[-1]
    ho, wo = h // stride, w // stride
    proj = sw is not None
    fin = post_bn is not None

    cpi, cpo = _cpad(cin), _cpad(cout)
    inputs = [x, s1.reshape(1, cin), t1.reshape(1, cin),
              _prep_w(w1, cin, cout),
              s2.reshape(1, cout), t2.reshape(1, cout),
              _prep_w(w2, cout, cout)]
    in_specs = [
        pl.BlockSpec((bb, h, w, cin), lambda i: (i, 0, 0, 0)),
        pl.BlockSpec((1, cin), lambda i: (0, 0)),
        pl.BlockSpec((1, cin), lambda i: (0, 0)),
        pl.BlockSpec((9 * cpi, cout), lambda i: (0, 0)),
        pl.BlockSpec((1, cout), lambda i: (0, 0)),
        pl.BlockSpec((1, cout), lambda i: (0, 0)),
        pl.BlockSpec((9 * cpo, cout), lambda i: (0, 0)),
    ]
    if proj:
        inputs.append(sw.astype(BF))
        in_specs.append(pl.BlockSpec((cin, cout), lambda i: (0, 0)))
    if fin:
        fs, ft = post_bn
        inputs += [fs.reshape(1, cout), ft.reshape(1, cout)]
        in_specs += [pl.BlockSpec((1, cout), lambda i: (0, 0)),
                     pl.BlockSpec((1, cout), lambda i: (0, 0))]

    flops = 2 * n * ho * wo * 9 * cout * (cin + cout)
    if proj:
        flops += 2 * n * ho * wo * cin * cout
    bytes_accessed = (n * h * w * cin + n * ho * wo * cout) * 4 + sum(
        int(a.size) * a.dtype.itemsize for a in inputs[1:])

    return pl.pallas_call(
        _make_block_body(bb, h, w, cin, cout, stride, proj, fin),
        out_shape=jax.ShapeDtypeStruct((n, ho, wo, cout), jnp.float32),
        grid=(n // bb,),
        in_specs=in_specs,
        out_specs=pl.BlockSpec((bb, ho, wo, cout), lambda i: (i, 0, 0, 0)),
        scratch_shapes=[pltpu.VMEM((bb, h + 2, w + 2, cin), BF),
                        pltpu.VMEM((bb, ho + 2, wo + 2, cout), BF),
                        pltpu.VMEM((bb * ho * wo, 9 * max(cpi, cpo)), BF)],
        compiler_params=pltpu.CompilerParams(
            dimension_semantics=("parallel",)),
        cost_estimate=pl.CostEstimate(flops=flops, transcendentals=0,
                                      bytes_accessed=bytes_accessed),
    )(*inputs)


def _make_stem_body(b, h, w, cin, cout):
    def body(x_ref, w_ref, o_ref, pad):
        _zero_border(pad, b, h + 2, w + 2, cin, jnp.float32)
        pad[:, 1:h + 1, 1:w + 1, :] = x_ref[...]
        m = b * h * w
        acc = jnp.zeros((m, cout), jnp.float32)
        for ki in range(3):
            for kj in range(3):
                win = pad[:, ki:ki + h, kj:kj + w, :]
                acc = acc + jnp.dot(win.reshape(m, cin), w_ref[ki, kj],
                                    preferred_element_type=jnp.float32)
        o_ref[...] = acc.reshape(b, h, w, cout)
    return body


def _stem(x, w, bb):
    n, h, wd, cin = x.shape
    cout = w.shape[-1]
    return pl.pallas_call(
        _make_stem_body(bb, h, wd, cin, cout),
        out_shape=jax.ShapeDtypeStruct((n, h, wd, cout), jnp.float32),
        grid=(n // bb,),
        in_specs=[pl.BlockSpec((bb, h, wd, cin), lambda i: (i, 0, 0, 0)),
                  pl.BlockSpec((3, 3, cin, cout), lambda i: (0, 0, 0, 0))],
        out_specs=pl.BlockSpec((bb, h, wd, cout), lambda i: (i, 0, 0, 0)),
        scratch_shapes=[pltpu.VMEM((bb, h + 2, wd + 2, cin), jnp.float32)],
        compiler_params=pltpu.CompilerParams(
            dimension_semantics=("parallel",)),
        cost_estimate=pl.CostEstimate(
            flops=2 * n * h * wd * 9 * cin * cout, transcendentals=0,
            bytes_accessed=(n * h * wd * (cin + cout)) * 4),
    )(x, w)


def kernel(x, stem_conv1_w, b0_bn1_scale, b0_bn1_shift, b0_conv1_w, b0_bn2_scale, b0_bn2_shift, b0_conv2_w, b0_shortcut_w, b1_bn1_scale, b1_bn1_shift, b1_conv1_w, b1_bn2_scale, b1_bn2_shift, b1_conv2_w, b2_bn1_scale, b2_bn1_shift, b2_conv1_w, b2_bn2_scale, b2_bn2_shift, b2_conv2_w, b3_bn1_scale, b3_bn1_shift, b3_conv1_w, b3_bn2_scale, b3_bn2_shift, b3_conv2_w, b4_bn1_scale, b4_bn1_shift, b4_conv1_w, b4_bn2_scale, b4_bn2_shift, b4_conv2_w, b4_shortcut_w, b5_bn1_scale, b5_bn1_shift, b5_conv1_w, b5_bn2_scale, b5_bn2_shift, b5_conv2_w, b6_bn1_scale, b6_bn1_shift, b6_conv1_w, b6_bn2_scale, b6_bn2_shift, b6_conv2_w, b7_bn1_scale, b7_bn1_shift, b7_conv1_w, b7_bn2_scale, b7_bn2_shift, b7_conv2_w, b8_bn1_scale, b8_bn1_shift, b8_conv1_w, b8_bn2_scale, b8_bn2_shift, b8_conv2_w, b8_shortcut_w, b9_bn1_scale, b9_bn1_shift, b9_conv1_w, b9_bn2_scale, b9_bn2_shift, b9_conv2_w, b10_bn1_scale, b10_bn1_shift, b10_conv1_w, b10_bn2_scale, b10_bn2_shift, b10_conv2_w, b11_bn1_scale, b11_bn1_shift, b11_conv1_w, b11_bn2_scale, b11_bn2_shift, b11_conv2_w, bn_final_scale, bn_final_shift):
    blocks = [
        (b0_bn1_scale, b0_bn1_shift, b0_conv1_w, b0_bn2_scale, b0_bn2_shift, b0_conv2_w, b0_shortcut_w, 1, 4),
        (b1_bn1_scale, b1_bn1_shift, b1_conv1_w, b1_bn2_scale, b1_bn2_shift, b1_conv2_w, None, 1, 4),
        (b2_bn1_scale, b2_bn1_shift, b2_conv1_w, b2_bn2_scale, b2_bn2_shift, b2_conv2_w, None, 1, 4),
        (b3_bn1_scale, b3_bn1_shift, b3_conv1_w, b3_bn2_scale, b3_bn2_shift, b3_conv2_w, None, 1, 4),
        (b4_bn1_scale, b4_bn1_shift, b4_conv1_w, b4_bn2_scale, b4_bn2_shift, b4_conv2_w, b4_shortcut_w, 2, 8),
        (b5_bn1_scale, b5_bn1_shift, b5_conv1_w, b5_bn2_scale, b5_bn2_shift, b5_conv2_w, None, 1, 8),
        (b6_bn1_scale, b6_bn1_shift, b6_conv1_w, b6_bn2_scale, b6_bn2_shift, b6_conv2_w, None, 1, 8),
        (b7_bn1_scale, b7_bn1_shift, b7_conv1_w, b7_bn2_scale, b7_bn2_shift, b7_conv2_w, None, 1, 8),
        (b8_bn1_scale, b8_bn1_shift, b8_conv1_w, b8_bn2_scale, b8_bn2_shift, b8_conv2_w, b8_shortcut_w, 2, 8),
        (b9_bn1_scale, b9_bn1_shift, b9_conv1_w, b9_bn2_scale, b9_bn2_shift, b9_conv2_w, None, 1, 8),
        (b10_bn1_scale, b10_bn1_shift, b10_conv1_w, b10_bn2_scale, b10_bn2_shift, b10_conv2_w, None, 1, 8),
        (b11_bn1_scale, b11_bn1_shift, b11_conv1_w, b11_bn2_scale, b11_bn2_shift, b11_conv2_w, None, 1, 8),
    ]
    h = jnp.transpose(x, (0, 2, 3, 1))
    h = _stem(h, stem_conv1_w, 8)
    last = len(blocks) - 1
    for i, (s1, t1, w1, s2, t2, w2, sw, stride, bb) in enumerate(blocks):
        post_bn = (bn_final_scale, bn_final_shift) if i == last else None
        h = _res_block(h, s1, t1, w1, s2, t2, w2, sw, stride, bb, post_bn)
    return jnp.transpose(h, (0, 3, 1, 2))


# speedup vs baseline: 3.9032x; 1.1516x over previous
import jax
import jax.numpy as jnp
from jax.experimental import pallas as pl
from jax.experimental.pallas import tpu as pltpu

BF = jnp.bfloat16


def _zero_border(ref, b, hp, wp, c, dtype):
    zr = jnp.zeros((b, 1, wp, c), dtype)
    ref[:, 0:1, :, :] = zr
    ref[:, hp - 1:hp, :, :] = zr
    zc = jnp.zeros((b, hp, 1, c), dtype)
    ref[:, :, 0:1, :] = zc
    ref[:, :, wp - 1:wp, :] = zc


def _cpad(c):
    return -(-c // 128) * 128


def _conv9(pad_ref, w_ref, pat_ref, b, ho, wo, cin, cout):
    m = b * ho * wo
    cp = _cpad(cin)
    kk = 9 * cp
    if cp > cin:
        zpad = jnp.zeros((m, cp - cin), BF)
        for t in range(9):
            pat_ref[:, t * cp + cin:(t + 1) * cp] = zpad
    t = 0
    for ki in range(3):
        for kj in range(3):
            win = pad_ref[:, ki:ki + ho, kj:kj + wo, :]
            pat_ref[:, t * cp:t * cp + cin] = win.reshape(m, cin)
            t += 1
    return jnp.dot(pat_ref[:, 0:kk], w_ref[...],
                   preferred_element_type=jnp.float32)


def _decimate2(v, b, ho, wo, c, ri=0, ci=0):
    v = v.reshape(b, ho, 2, 2 * wo, c)[:, :, ri]
    return v.reshape(b, ho, wo, 2, c)[:, :, :, ci]


def _conv9_s2(a1, w_ref, pat_ref, ph_refs, b, ho, wo, cin, cout):
    m = b * ho * wo
    cp = _cpad(cin)
    kk = 9 * cp
    p00, p01, p10, p11 = ph_refs
    zrow = jnp.zeros((b, 1, wo + 1, cin), BF)
    zcol = jnp.zeros((b, ho + 1, 1, cin), BF)
    p00[:, 0:1, :, :] = zrow
    p00[:, :, 0:1, :] = zcol
    p00[:, 1:ho + 1, 1:wo + 1, :] = _decimate2(a1, b, ho, wo, cin, 1, 1)
    p01[:, 0:1, :, :] = zrow
    p01[:, :, wo:wo + 1, :] = zcol
    p01[:, 1:ho + 1, 0:wo, :] = _decimate2(a1, b, ho, wo, cin, 1, 0)
    p10[:, ho:ho + 1, :, :] = zrow
    p10[:, :, 0:1, :] = zcol
    p10[:, 0:ho, 1:wo + 1, :] = _decimate2(a1, b, ho, wo, cin, 0, 1)
    p11[:, ho:ho + 1, :, :] = zrow
    p11[:, :, wo:wo + 1, :] = zcol
    p11[:, 0:ho, 0:wo, :] = _decimate2(a1, b, ho, wo, cin, 0, 0)
    phases = ((p00, p01), (p10, p11))

    if cp > cin:
        zpad = jnp.zeros((m, cp - cin), BF)
        for t in range(9):
            pat_ref[:, t * cp + cin:(t + 1) * cp] = zpad
    t = 0
    for ki in range(3):
        for kj in range(3):
            ph = phases[ki & 1][kj & 1]
            oi, oj = ki // 2, kj // 2
            win = ph[:, oi:oi + ho, oj:oj + wo, :]
            pat_ref[:, t * cp:t * cp + cin] = win.reshape(m, cin)
            t += 1
    return jnp.dot(pat_ref[:, 0:kk], w_ref[...],
                   preferred_element_type=jnp.float32)


def _prep_w(w, cin, cout):
    cp = _cpad(cin)
    wp = jnp.pad(w.astype(BF), ((0, 0), (0, 0), (0, cp - cin), (0, 0)))
    return wp.reshape(9 * cp, cout)


def _make_block_body(b, h, w, cin, cout, stride, proj, fin):
    ho, wo = h // stride, w // stride
    m = b * ho * wo

    def body(*refs):
        it = iter(refs)
        x_ref = next(it)
        s1_ref = next(it)
        t1_ref = next(it)
        w1_ref = next(it)
        s2_ref = next(it)
        t2_ref = next(it)
        w2_ref = next(it)
        sw_ref = next(it) if proj else None
        if fin:
            fs_ref = next(it)
            ft_ref = next(it)
        o_ref = next(it)
        if stride == 1:
            pad1 = next(it)
        pad2 = next(it)
        pat = next(it)
        if stride == 2:
            ph_refs = (next(it), next(it), next(it), next(it))

        x = x_ref[...]
        a1 = jnp.maximum(x * s1_ref[...] + t1_ref[...], 0.0).astype(BF)

        if stride == 1:
            _zero_border(pad1, b, h + 2, w + 2, cin, BF)
            pad1[:, 1:h + 1, 1:w + 1, :] = a1
            acc1 = _conv9(pad1, w1_ref, pat, b, ho, wo, cin, cout)
        else:
            acc1 = _conv9_s2(a1, w1_ref, pat, ph_refs, b, ho, wo, cin, cout)

        a2 = jnp.maximum(acc1 * s2_ref[...] + t2_ref[...], 0.0)
        _zero_border(pad2, b, ho + 2, wo + 2, cout, BF)
        pad2[:, 1:ho + 1, 1:wo + 1, :] = a2.astype(BF).reshape(b, ho, wo, cout)

        acc2 = _conv9(pad2, w2_ref, pat, b, ho, wo, cout, cout)

        if proj:
            xs = _decimate2(x, b, ho, wo, cin) if stride == 2 else x
            acc2 = acc2 + jnp.dot(xs.reshape(m, cin).astype(BF), sw_ref[...],
                                  preferred_element_type=jnp.float32)
        else:
            acc2 = acc2 + x.reshape(m, cout)

        if fin:
            acc2 = jnp.maximum(acc2 * fs_ref[...] + ft_ref[...], 0.0)

        o_ref[...] = acc2.reshape(b, ho, wo, cout)

    return body


def _res_block(x, s1, t1, w1, s2, t2, w2, sw, stride, bb, post_bn):
    n, h, w, cin = x.shape
    cout = w1.shape[-1]
    ho, wo = h // stride, w // stride
    proj = sw is not None
    fin = post_bn is not None

    cpi, cpo = _cpad(cin), _cpad(cout)
    inputs = [x, s1.reshape(1, cin), t1.reshape(1, cin),
              _prep_w(w1, cin, cout),
              s2.reshape(1, cout), t2.reshape(1, cout),
              _prep_w(w2, cout, cout)]
    in_specs = [
        pl.BlockSpec((bb, h, w, cin), lambda i: (i, 0, 0, 0)),
        pl.BlockSpec((1, cin), lambda i: (0, 0)),
        pl.BlockSpec((1, cin), lambda i: (0, 0)),
        pl.BlockSpec((9 * cpi, cout), lambda i: (0, 0)),
        pl.BlockSpec((1, cout), lambda i: (0, 0)),
        pl.BlockSpec((1, cout), lambda i: (0, 0)),
        pl.BlockSpec((9 * cpo, cout), lambda i: (0, 0)),
    ]
    if proj:
        inputs.append(sw.astype(BF))
        in_specs.append(pl.BlockSpec((cin, cout), lambda i: (0, 0)))
    if fin:
        fs, ft = post_bn
        inputs += [fs.reshape(1, cout), ft.reshape(1, cout)]
        in_specs += [pl.BlockSpec((1, cout), lambda i: (0, 0)),
                     pl.BlockSpec((1, cout), lambda i: (0, 0))]

    flops = 2 * n * ho * wo * 9 * cout * (cin + cout)
    if proj:
        flops += 2 * n * ho * wo * cin * cout
    bytes_accessed = (n * h * w * cin + n * ho * wo * cout) * 4 + sum(
        int(a.size) * a.dtype.itemsize for a in inputs[1:])

    scratch = ([pltpu.VMEM((bb, h + 2, w + 2, cin), BF)] if stride == 1 else [])
    scratch += [pltpu.VMEM((bb, ho + 2, wo + 2, cout), BF),
                pltpu.VMEM((bb * ho * wo, 9 * max(cpi, cpo)), BF)]
    if stride == 2:
        scratch += [pltpu.VMEM((bb, ho + 1, wo + 1, cin), BF)] * 4

    return pl.pallas_call(
        _make_block_body(bb, h, w, cin, cout, stride, proj, fin),
        out_shape=jax.ShapeDtypeStruct((n, ho, wo, cout), jnp.float32),
        grid=(n // bb,),
        in_specs=in_specs,
        out_specs=pl.BlockSpec((bb, ho, wo, cout), lambda i: (i, 0, 0, 0)),
        scratch_shapes=scratch,
        compiler_params=pltpu.CompilerParams(
            dimension_semantics=("parallel",)),
        cost_estimate=pl.CostEstimate(flops=flops, transcendentals=0,
                                      bytes_accessed=bytes_accessed),
    )(*inputs)


def _make_stem_body(b, h, w, cin, cout):
    def body(x_ref, w_ref, o_ref, pad):
        _zero_border(pad, b, h + 2, w + 2, cin, jnp.float32)
        pad[:, 1:h + 1, 1:w + 1, :] = x_ref[...]
        m = b * h * w
        acc = jnp.zeros((m, cout), jnp.float32)
        for ki in range(3):
            for kj in range(3):
                win = pad[:, ki:ki + h, kj:kj + w, :]
                acc = acc + jnp.dot(win.reshape(m, cin), w_ref[ki, kj],
                                    preferred_element_type=jnp.float32)
        o_ref[...] = acc.reshape(b, h, w, cout)
    return body


def _stem(x, w, bb):
    n, h, wd, cin = x.shape
    cout = w.shape[-1]
    return pl.pallas_call(
        _make_stem_body(bb, h, wd, cin, cout),
        out_shape=jax.ShapeDtypeStruct((n, h, wd, cout), jnp.float32),
        grid=(n // bb,),
        in_specs=[pl.BlockSpec((bb, h, wd, cin), lambda i: (i, 0, 0, 0)),
                  pl.BlockSpec((3, 3, cin, cout), lambda i: (0, 0, 0, 0))],
        out_specs=pl.BlockSpec((bb, h, wd, cout), lambda i: (i, 0, 0, 0)),
        scratch_shapes=[pltpu.VMEM((bb, h + 2, wd + 2, cin), jnp.float32)],
        compiler_params=pltpu.CompilerParams(
            dimension_semantics=("parallel",)),
        cost_estimate=pl.CostEstimate(
            flops=2 * n * h * wd * 9 * cin * cout, transcendentals=0,
            bytes_accessed=(n * h * wd * (cin + cout)) * 4),
    )(x, w)


def kernel(x, stem_conv1_w, b0_bn1_scale, b0_bn1_shift, b0_conv1_w, b0_bn2_scale, b0_bn2_shift, b0_conv2_w, b0_shortcut_w, b1_bn1_scale, b1_bn1_shift, b1_conv1_w, b1_bn2_scale, b1_bn2_shift, b1_conv2_w, b2_bn1_scale, b2_bn1_shift, b2_conv1_w, b2_bn2_scale, b2_bn2_shift, b2_conv2_w, b3_bn1_scale, b3_bn1_shift, b3_conv1_w, b3_bn2_scale, b3_bn2_shift, b3_conv2_w, b4_bn1_scale, b4_bn1_shift, b4_conv1_w, b4_bn2_scale, b4_bn2_shift, b4_conv2_w, b4_shortcut_w, b5_bn1_scale, b5_bn1_shift, b5_conv1_w, b5_bn2_scale, b5_bn2_shift, b5_conv2_w, b6_bn1_scale, b6_bn1_shift, b6_conv1_w, b6_bn2_scale, b6_bn2_shift, b6_conv2_w, b7_bn1_scale, b7_bn1_shift, b7_conv1_w, b7_bn2_scale, b7_bn2_shift, b7_conv2_w, b8_bn1_scale, b8_bn1_shift, b8_conv1_w, b8_bn2_scale, b8_bn2_shift, b8_conv2_w, b8_shortcut_w, b9_bn1_scale, b9_bn1_shift, b9_conv1_w, b9_bn2_scale, b9_bn2_shift, b9_conv2_w, b10_bn1_scale, b10_bn1_shift, b10_conv1_w, b10_bn2_scale, b10_bn2_shift, b10_conv2_w, b11_bn1_scale, b11_bn1_shift, b11_conv1_w, b11_bn2_scale, b11_bn2_shift, b11_conv2_w, bn_final_scale, bn_final_shift):
    blocks = [
        (b0_bn1_scale, b0_bn1_shift, b0_conv1_w, b0_bn2_scale, b0_bn2_shift, b0_conv2_w, b0_shortcut_w, 1, 4),
        (b1_bn1_scale, b1_bn1_shift, b1_conv1_w, b1_bn2_scale, b1_bn2_shift, b1_conv2_w, None, 1, 4),
        (b2_bn1_scale, b2_bn1_shift, b2_conv1_w, b2_bn2_scale, b2_bn2_shift, b2_conv2_w, None, 1, 4),
        (b3_bn1_scale, b3_bn1_shift, b3_conv1_w, b3_bn2_scale, b3_bn2_shift, b3_conv2_w, None, 1, 4),
        (b4_bn1_scale, b4_bn1_shift, b4_conv1_w, b4_bn2_scale, b4_bn2_shift, b4_conv2_w, b4_shortcut_w, 2, 4),
        (b5_bn1_scale, b5_bn1_shift, b5_conv1_w, b5_bn2_scale, b5_bn2_shift, b5_conv2_w, None, 1, 8),
        (b6_bn1_scale, b6_bn1_shift, b6_conv1_w, b6_bn2_scale, b6_bn2_shift, b6_conv2_w, None, 1, 8),
        (b7_bn1_scale, b7_bn1_shift, b7_conv1_w, b7_bn2_scale, b7_bn2_shift, b7_conv2_w, None, 1, 8),
        (b8_bn1_scale, b8_bn1_shift, b8_conv1_w, b8_bn2_scale, b8_bn2_shift, b8_conv2_w, b8_shortcut_w, 2, 4),
        (b9_bn1_scale, b9_bn1_shift, b9_conv1_w, b9_bn2_scale, b9_bn2_shift, b9_conv2_w, None, 1, 8),
        (b10_bn1_scale, b10_bn1_shift, b10_conv1_w, b10_bn2_scale, b10_bn2_shift, b10_conv2_w, None, 1, 8),
        (b11_bn1_scale, b11_bn1_shift, b11_conv1_w, b11_bn2_scale, b11_bn2_shift, b11_conv2_w, None, 1, 8),
    ]
    h = jnp.transpose(x, (0, 2, 3, 1))
    h = _stem(h, stem_conv1_w, 8)
    last = len(blocks) - 1
    for i, (s1, t1, w1, s2, t2, w2, sw, stride, bb) in enumerate(blocks):
        post_bn = (bn_final_scale, bn_final_shift) if i == last else None
        h = _res_block(h, s1, t1, w1, s2, t2, w2, sw, stride, bb, post_bn)
    return jnp.transpose(h, (0, 3, 1, 2))


# speedup vs baseline: 4.6776x; 1.1984x over previous
import jax
import jax.numpy as jnp
from jax.experimental import pallas as pl
from jax.experimental.pallas import tpu as pltpu

BF = jnp.bfloat16


def _zero_border(ref, b, hp, wp, c, dtype):
    zr = jnp.zeros((b, 1, wp, c), dtype)
    ref[:, 0:1, :, :] = zr
    ref[:, hp - 1:hp, :, :] = zr
    zc = jnp.zeros((b, hp, 1, c), dtype)
    ref[:, :, 0:1, :] = zc
    ref[:, :, wp - 1:wp, :] = zc


def _cpad(c):
    return c if c >= 128 else 128


def _conv9(pad_ref, w_ref, pat_ref, b, ho, wo, cin, cout):
    m = b * ho * wo
    cp = _cpad(cin)
    kk = 9 * cp
    if cp > cin:
        zpad = jnp.zeros((m, cp - cin), BF)
        for t in range(9):
            pat_ref[:, t * cp + cin:(t + 1) * cp] = zpad
    t = 0
    for ki in range(3):
        for kj in range(3):
            win = pad_ref[:, ki:ki + ho, kj:kj + wo, :]
            pat_ref[:, t * cp:t * cp + cin] = win.reshape(m, cin)
            t += 1
    return jnp.dot(pat_ref[:, 0:kk], w_ref[...],
                   preferred_element_type=jnp.float32)


def _decimate2(v, b, ho, wo, c, ri=0, ci=0):
    v = v.reshape(b, ho, 2, 2 * wo, c)[:, :, ri]
    return v.reshape(b, ho, wo, 2, c)[:, :, :, ci]


def _conv9_s2(a1, w_ref, pat_ref, ph_refs, b, ho, wo, cin, cout):
    m = b * ho * wo
    cp = _cpad(cin)
    kk = 9 * cp
    p00, p01, p10, p11 = ph_refs
    zrow = jnp.zeros((b, 1, wo + 1, cin), BF)
    zcol = jnp.zeros((b, ho + 1, 1, cin), BF)
    p00[:, 0:1, :, :] = zrow
    p00[:, :, 0:1, :] = zcol
    p00[:, 1:ho + 1, 1:wo + 1, :] = _decimate2(a1, b, ho, wo, cin, 1, 1)
    p01[:, 0:1, :, :] = zrow
    p01[:, :, wo:wo + 1, :] = zcol
    p01[:, 1:ho + 1, 0:wo, :] = _decimate2(a1, b, ho, wo, cin, 1, 0)
    p10[:, ho:ho + 1, :, :] = zrow
    p10[:, :, 0:1, :] = zcol
    p10[:, 0:ho, 1:wo + 1, :] = _decimate2(a1, b, ho, wo, cin, 0, 1)
    p11[:, ho:ho + 1, :, :] = zrow
    p11[:, :, wo:wo + 1, :] = zcol
    p11[:, 0:ho, 0:wo, :] = _decimate2(a1, b, ho, wo, cin, 0, 0)
    phases = ((p00, p01), (p10, p11))

    if cp > cin:
        zpad = jnp.zeros((m, cp - cin), BF)
        for t in range(9):
            pat_ref[:, t * cp + cin:(t + 1) * cp] = zpad
    t = 0
    for ki in range(3):
        for kj in range(3):
            ph = phases[ki & 1][kj & 1]
            oi, oj = ki // 2, kj // 2
            win = ph[:, oi:oi + ho, oj:oj + wo, :]
            pat_ref[:, t * cp:t * cp + cin] = win.reshape(m, cin)
            t += 1
    return jnp.dot(pat_ref[:, 0:kk], w_ref[...],
                   preferred_element_type=jnp.float32)


def _prep_w(w, cin, cout):
    cp = _cpad(cin)
    wp = jnp.pad(w.astype(BF), ((0, 0), (0, 0), (0, cp - cin), (0, 0)))
    return wp.reshape(9 * cp, cout)


def _make_block_body(b, h, w, cin, cout, stride, proj, fin):
    ho, wo = h // stride, w // stride
    m = b * ho * wo

    def body(*refs):
        it = iter(refs)
        x_ref = next(it)
        s1_ref = next(it)
        t1_ref = next(it)
        w1_ref = next(it)
        s2_ref = next(it)
        t2_ref = next(it)
        w2_ref = next(it)
        sw_ref = next(it) if proj else None
        if fin:
            fs_ref = next(it)
            ft_ref = next(it)
        o_ref = next(it)
        if stride == 1:
            pad1 = next(it)
        pad2 = next(it)
        pat = next(it)
        if stride == 2:
            ph_refs = (next(it), next(it), next(it), next(it))

        x = x_ref[...]
        a1 = jnp.maximum(x * s1_ref[...] + t1_ref[...], 0.0).astype(BF)

        if stride == 1:
            _zero_border(pad1, b, h + 2, w + 2, cin, BF)
            pad1[:, 1:h + 1, 1:w + 1, :] = a1
            acc1 = _conv9(pad1, w1_ref, pat, b, ho, wo, cin, cout)
        else:
            acc1 = _conv9_s2(a1, w1_ref, pat, ph_refs, b, ho, wo, cin, cout)

        a2 = jnp.maximum(acc1 * s2_ref[...] + t2_ref[...], 0.0)
        _zero_border(pad2, b, ho + 2, wo + 2, cout, BF)
        pad2[:, 1:ho + 1, 1:wo + 1, :] = a2.astype(BF).reshape(b, ho, wo, cout)

        acc2 = _conv9(pad2, w2_ref, pat, b, ho, wo, cout, cout)

        if proj:
            xs = _decimate2(x, b, ho, wo, cin) if stride == 2 else x
            acc2 = acc2 + jnp.dot(xs.reshape(m, cin).astype(BF), sw_ref[...],
                                  preferred_element_type=jnp.float32)
        else:
            acc2 = acc2 + x.reshape(m, cout).astype(jnp.float32)

        if fin:
            acc2 = jnp.maximum(acc2 * fs_ref[...] + ft_ref[...], 0.0)

        o_ref[...] = acc2.reshape(b, ho, wo, cout).astype(o_ref.dtype)

    return body


def _res_block(x, s1, t1, w1, s2, t2, w2, sw, stride, bb, post_bn,
               out_dtype=BF):
    n, h, w, cin = x.shape
    cout = w1.shape[-1]
    ho, wo = h // stride, w // stride
    proj = sw is not None
    fin = post_bn is not None

    cpi, cpo = _cpad(cin), _cpad(cout)
    inputs = [x, s1.reshape(1, cin), t1.reshape(1, cin),
              _prep_w(w1, cin, cout),
              s2.reshape(1, cout), t2.reshape(1, cout),
              _prep_w(w2, cout, cout)]
    in_specs = [
        pl.BlockSpec((bb, h, w, cin), lambda i: (i, 0, 0, 0)),
        pl.BlockSpec((1, cin), lambda i: (0, 0)),
        pl.BlockSpec((1, cin), lambda i: (0, 0)),
        pl.BlockSpec((9 * cpi, cout), lambda i: (0, 0)),
        pl.BlockSpec((1, cout), lambda i: (0, 0)),
        pl.BlockSpec((1, cout), lambda i: (0, 0)),
        pl.BlockSpec((9 * cpo, cout), lambda i: (0, 0)),
    ]
    if proj:
        inputs.append(sw.astype(BF))
        in_specs.append(pl.BlockSpec((cin, cout), lambda i: (0, 0)))
    if fin:
        fs, ft = post_bn
        inputs += [fs.reshape(1, cout), ft.reshape(1, cout)]
        in_specs += [pl.BlockSpec((1, cout), lambda i: (0, 0)),
                     pl.BlockSpec((1, cout), lambda i: (0, 0))]

    flops = 2 * n * ho * wo * 9 * cout * (cin + cout)
    if proj:
        flops += 2 * n * ho * wo * cin * cout
    bytes_accessed = (n * h * w * cin + n * ho * wo * cout) * 4 + sum(
        int(a.size) * a.dtype.itemsize for a in inputs[1:])

    scratch = ([pltpu.VMEM((bb, h + 2, w + 2, cin), BF)] if stride == 1 else [])
    scratch += [pltpu.VMEM((bb, ho + 2, wo + 2, cout), BF),
                pltpu.VMEM((bb * ho * wo, 9 * max(cpi, cpo)), BF)]
    if stride == 2:
        scratch += [pltpu.VMEM((bb, ho + 1, wo + 1, cin), BF)] * 4

    return pl.pallas_call(
        _make_block_body(bb, h, w, cin, cout, stride, proj, fin),
        out_shape=jax.ShapeDtypeStruct((n, ho, wo, cout), out_dtype),
        grid=(n // bb,),
        in_specs=in_specs,
        out_specs=pl.BlockSpec((bb, ho, wo, cout), lambda i: (i, 0, 0, 0)),
        scratch_shapes=scratch,
        compiler_params=pltpu.CompilerParams(
            dimension_semantics=("parallel",)),
        cost_estimate=pl.CostEstimate(flops=flops, transcendentals=0,
                                      bytes_accessed=bytes_accessed),
    )(*inputs)


def _make_stem_body(b, h, w, cin, cout):
    def body(x_ref, w_ref, o_ref, pad):
        _zero_border(pad, b, h + 2, w + 2, cin, jnp.float32)
        pad[:, 1:h + 1, 1:w + 1, :] = x_ref[...]
        m = b * h * w
        acc = jnp.zeros((m, cout), jnp.float32)
        for ki in range(3):
            for kj in range(3):
                win = pad[:, ki:ki + h, kj:kj + w, :]
                acc = acc + jnp.dot(win.reshape(m, cin), w_ref[ki, kj],
                                    preferred_element_type=jnp.float32)
        o_ref[...] = acc.reshape(b, h, w, cout).astype(o_ref.dtype)
    return body


def _stem(x, w, bb):
    n, h, wd, cin = x.shape
    cout = w.shape[-1]
    return pl.pallas_call(
        _make_stem_body(bb, h, wd, cin, cout),
        out_shape=jax.ShapeDtypeStruct((n, h, wd, cout), BF),
        grid=(n // bb,),
        in_specs=[pl.BlockSpec((bb, h, wd, cin), lambda i: (i, 0, 0, 0)),
                  pl.BlockSpec((3, 3, cin, cout), lambda i: (0, 0, 0, 0))],
        out_specs=pl.BlockSpec((bb, h, wd, cout), lambda i: (i, 0, 0, 0)),
        scratch_shapes=[pltpu.VMEM((bb, h + 2, wd + 2, cin), jnp.float32)],
        compiler_params=pltpu.CompilerParams(
            dimension_semantics=("parallel",)),
        cost_estimate=pl.CostEstimate(
            flops=2 * n * h * wd * 9 * cin * cout, transcendentals=0,
            bytes_accessed=(n * h * wd * (cin + cout)) * 4),
    )(x, w)


def kernel(x, stem_conv1_w, b0_bn1_scale, b0_bn1_shift, b0_conv1_w, b0_bn2_scale, b0_bn2_shift, b0_conv2_w, b0_shortcut_w, b1_bn1_scale, b1_bn1_shift, b1_conv1_w, b1_bn2_scale, b1_bn2_shift, b1_conv2_w, b2_bn1_scale, b2_bn1_shift, b2_conv1_w, b2_bn2_scale, b2_bn2_shift, b2_conv2_w, b3_bn1_scale, b3_bn1_shift, b3_conv1_w, b3_bn2_scale, b3_bn2_shift, b3_conv2_w, b4_bn1_scale, b4_bn1_shift, b4_conv1_w, b4_bn2_scale, b4_bn2_shift, b4_conv2_w, b4_shortcut_w, b5_bn1_scale, b5_bn1_shift, b5_conv1_w, b5_bn2_scale, b5_bn2_shift, b5_conv2_w, b6_bn1_scale, b6_bn1_shift, b6_conv1_w, b6_bn2_scale, b6_bn2_shift, b6_conv2_w, b7_bn1_scale, b7_bn1_shift, b7_conv1_w, b7_bn2_scale, b7_bn2_shift, b7_conv2_w, b8_bn1_scale, b8_bn1_shift, b8_conv1_w, b8_bn2_scale, b8_bn2_shift, b8_conv2_w, b8_shortcut_w, b9_bn1_scale, b9_bn1_shift, b9_conv1_w, b9_bn2_scale, b9_bn2_shift, b9_conv2_w, b10_bn1_scale, b10_bn1_shift, b10_conv1_w, b10_bn2_scale, b10_bn2_shift, b10_conv2_w, b11_bn1_scale, b11_bn1_shift, b11_conv1_w, b11_bn2_scale, b11_bn2_shift, b11_conv2_w, bn_final_scale, bn_final_shift):
    blocks = [
        (b0_bn1_scale, b0_bn1_shift, b0_conv1_w, b0_bn2_scale, b0_bn2_shift, b0_conv2_w, b0_shortcut_w, 1, 4),
        (b1_bn1_scale, b1_bn1_shift, b1_conv1_w, b1_bn2_scale, b1_bn2_shift, b1_conv2_w, None, 1, 4),
        (b2_bn1_scale, b2_bn1_shift, b2_conv1_w, b2_bn2_scale, b2_bn2_shift, b2_conv2_w, None, 1, 4),
        (b3_bn1_scale, b3_bn1_shift, b3_conv1_w, b3_bn2_scale, b3_bn2_shift, b3_conv2_w, None, 1, 4),
        (b4_bn1_scale, b4_bn1_shift, b4_conv1_w, b4_bn2_scale, b4_bn2_shift, b4_conv2_w, b4_shortcut_w, 2, 4),
        (b5_bn1_scale, b5_bn1_shift, b5_conv1_w, b5_bn2_scale, b5_bn2_shift, b5_conv2_w, None, 1, 8),
        (b6_bn1_scale, b6_bn1_shift, b6_conv1_w, b6_bn2_scale, b6_bn2_shift, b6_conv2_w, None, 1, 8),
        (b7_bn1_scale, b7_bn1_shift, b7_conv1_w, b7_bn2_scale, b7_bn2_shift, b7_conv2_w, None, 1, 8),
        (b8_bn1_scale, b8_bn1_shift, b8_conv1_w, b8_bn2_scale, b8_bn2_shift, b8_conv2_w, b8_shortcut_w, 2, 4),
        (b9_bn1_scale, b9_bn1_shift, b9_conv1_w, b9_bn2_scale, b9_bn2_shift, b9_conv2_w, None, 1, 8),
        (b10_bn1_scale, b10_bn1_shift, b10_conv1_w, b10_bn2_scale, b10_bn2_shift, b10_conv2_w, None, 1, 8),
        (b11_bn1_scale, b11_bn1_shift, b11_conv1_w, b11_bn2_scale, b11_bn2_shift, b11_conv2_w, None, 1, 8),
    ]
    h = jnp.transpose(x, (0, 2, 3, 1))
    h = _stem(h, stem_conv1_w, 8)
    last = len(blocks) - 1
    for i, (s1, t1, w1, s2, t2, w2, sw, stride, bb) in enumerate(blocks):
        post_bn = (bn_final_scale, bn_final_shift) if i == last else None
        odt = jnp.float32 if i == last else BF
        h = _res_block(h, s1, t1, w1, s2, t2, w2, sw, stride, bb, post_bn,
                       out_dtype=odt)
    return jnp.transpose(h, (0, 3, 1, 2))


# speedup vs baseline: 4.6904x; 1.0027x over previous
import jax
import jax.numpy as jnp
from jax.experimental import pallas as pl
from jax.experimental.pallas import tpu as pltpu

BF = jnp.bfloat16


def _zero_border(ref, b, hp, wp, c, dtype):
    zr = jnp.zeros((b, 1, wp, c), dtype)
    ref[:, 0:1, :, :] = zr
    ref[:, hp - 1:hp, :, :] = zr
    zc = jnp.zeros((b, hp, 1, c), dtype)
    ref[:, :, 0:1, :] = zc
    ref[:, :, wp - 1:wp, :] = zc


def _cpad(c):
    return c if c >= 128 else 128


def _conv9(pad_ref, w_ref, pat_ref, b, ho, wo, cin, cout):
    m = b * ho * wo
    cp = _cpad(cin)
    kk = 9 * cp
    if cp > cin:
        zpad = jnp.zeros((m, cp - cin), BF)
        for t in range(9):
            pat_ref[:, t * cp + cin:(t + 1) * cp] = zpad
    t = 0
    for ki in range(3):
        for kj in range(3):
            win = pad_ref[:, ki:ki + ho, kj:kj + wo, :]
            pat_ref[:, t * cp:t * cp + cin] = win.reshape(m, cin)
            t += 1
    return jnp.dot(pat_ref[:, 0:kk], w_ref[...],
                   preferred_element_type=jnp.float32)


def _decimate2(v, b, ho, wo, c, ri=0, ci=0):
    v = v.reshape(b, ho, 2, 2 * wo, c)[:, :, ri]
    return v.reshape(b, ho, wo, 2, c)[:, :, :, ci]


def _conv9_s2(a1, w_ref, pat_ref, ph_refs, b, ho, wo, cin, cout):
    m = b * ho * wo
    cp = _cpad(cin)
    kk = 9 * cp
    p00, p01, p10, p11 = ph_refs
    zrow = jnp.zeros((b, 1, wo + 1, cin), BF)
    zcol = jnp.zeros((b, ho + 1, 1, cin), BF)
    p00[:, 0:1, :, :] = zrow
    p00[:, :, 0:1, :] = zcol
    p00[:, 1:ho + 1, 1:wo + 1, :] = _decimate2(a1, b, ho, wo, cin, 1, 1)
    p01[:, 0:1, :, :] = zrow
    p01[:, :, wo:wo + 1, :] = zcol
    p01[:, 1:ho + 1, 0:wo, :] = _decimate2(a1, b, ho, wo, cin, 1, 0)
    p10[:, ho:ho + 1, :, :] = zrow
    p10[:, :, 0:1, :] = zcol
    p10[:, 0:ho, 1:wo + 1, :] = _decimate2(a1, b, ho, wo, cin, 0, 1)
    p11[:, ho:ho + 1, :, :] = zrow
    p11[:, :, wo:wo + 1, :] = zcol
    p11[:, 0:ho, 0:wo, :] = _decimate2(a1, b, ho, wo, cin, 0, 0)
    phases = ((p00, p01), (p10, p11))

    if cp > cin:
        zpad = jnp.zeros((m, cp - cin), BF)
        for t in range(9):
            pat_ref[:, t * cp + cin:(t + 1) * cp] = zpad
    t = 0
    for ki in range(3):
        for kj in range(3):
            ph = phases[ki & 1][kj & 1]
            oi, oj = ki // 2, kj // 2
            win = ph[:, oi:oi + ho, oj:oj + wo, :]
            pat_ref[:, t * cp:t * cp + cin] = win.reshape(m, cin)
            t += 1
    return jnp.dot(pat_ref[:, 0:kk], w_ref[...],
                   preferred_element_type=jnp.float32)


def _prep_w(w, cin, cout):
    cp = _cpad(cin)
    wp = jnp.pad(w.astype(BF), ((0, 0), (0, 0), (0, cp - cin), (0, 0)))
    return wp.reshape(9 * cp, cout)


def _make_block_body(b, h, w, cin, cout, stride, proj, fin):
    ho, wo = h // stride, w // stride
    m = b * ho * wo

    def body(*refs):
        it = iter(refs)
        x_ref = next(it)
        s1_ref = next(it)
        t1_ref = next(it)
        w1_ref = next(it)
        s2_ref = next(it)
        t2_ref = next(it)
        w2_ref = next(it)
        sw_ref = next(it) if proj else None
        if fin:
            fs_ref = next(it)
            ft_ref = next(it)
        o_ref = next(it)
        if stride == 1:
            pad1 = next(it)
        pad2 = next(it)
        pat = next(it)
        if stride == 2:
            ph_refs = (next(it), next(it), next(it), next(it))

        x = x_ref[...]
        a1 = jnp.maximum(x * s1_ref[...] + t1_ref[...], 0.0).astype(BF)

        if stride == 1:
            _zero_border(pad1, b, h + 2, w + 2, cin, BF)
            pad1[:, 1:h + 1, 1:w + 1, :] = a1
            acc1 = _conv9(pad1, w1_ref, pat, b, ho, wo, cin, cout)
        else:
            acc1 = _conv9_s2(a1, w1_ref, pat, ph_refs, b, ho, wo, cin, cout)

        a2 = jnp.maximum(acc1 * s2_ref[...] + t2_ref[...], 0.0)
        _zero_border(pad2, b, ho + 2, wo + 2, cout, BF)
        pad2[:, 1:ho + 1, 1:wo + 1, :] = a2.astype(BF).reshape(b, ho, wo, cout)

        acc2 = _conv9(pad2, w2_ref, pat, b, ho, wo, cout, cout)

        if proj:
            xs = _decimate2(x, b, ho, wo, cin) if stride == 2 else x
            acc2 = acc2 + jnp.dot(xs.reshape(m, cin).astype(BF), sw_ref[...],
                                  preferred_element_type=jnp.float32)
        else:
            acc2 = acc2 + x.reshape(m, cout).astype(jnp.float32)

        if fin:
            acc2 = jnp.maximum(acc2 * fs_ref[...] + ft_ref[...], 0.0)

        o_ref[...] = acc2.reshape(b, ho, wo, cout).astype(o_ref.dtype)

    return body


def _res_block(x, s1, t1, w1, s2, t2, w2, sw, stride, bb, post_bn,
               out_dtype=BF):
    n, h, w, cin = x.shape
    cout = w1.shape[-1]
    ho, wo = h // stride, w // stride
    proj = sw is not None
    fin = post_bn is not None

    cpi, cpo = _cpad(cin), _cpad(cout)
    inputs = [x, s1.reshape(1, cin), t1.reshape(1, cin),
              _prep_w(w1, cin, cout),
              s2.reshape(1, cout), t2.reshape(1, cout),
              _prep_w(w2, cout, cout)]
    in_specs = [
        pl.BlockSpec((bb, h, w, cin), lambda i: (i, 0, 0, 0)),
        pl.BlockSpec((1, cin), lambda i: (0, 0)),
        pl.BlockSpec((1, cin), lambda i: (0, 0)),
        pl.BlockSpec((9 * cpi, cout), lambda i: (0, 0)),
        pl.BlockSpec((1, cout), lambda i: (0, 0)),
        pl.BlockSpec((1, cout), lambda i: (0, 0)),
        pl.BlockSpec((9 * cpo, cout), lambda i: (0, 0)),
    ]
    if proj:
        inputs.append(sw.astype(BF))
        in_specs.append(pl.BlockSpec((cin, cout), lambda i: (0, 0)))
    if fin:
        fs, ft = post_bn
        inputs += [fs.reshape(1, cout), ft.reshape(1, cout)]
        in_specs += [pl.BlockSpec((1, cout), lambda i: (0, 0)),
                     pl.BlockSpec((1, cout), lambda i: (0, 0))]

    flops = 2 * n * ho * wo * 9 * cout * (cin + cout)
    if proj:
        flops += 2 * n * ho * wo * cin * cout
    bytes_accessed = (n * h * w * cin + n * ho * wo * cout) * 4 + sum(
        int(a.size) * a.dtype.itemsize for a in inputs[1:])

    scratch = ([pltpu.VMEM((bb, h + 2, w + 2, cin), BF)] if stride == 1 else [])
    scratch += [pltpu.VMEM((bb, ho + 2, wo + 2, cout), BF),
                pltpu.VMEM((bb * ho * wo, 9 * max(cpi, cpo)), BF)]
    if stride == 2:
        scratch += [pltpu.VMEM((bb, ho + 1, wo + 1, cin), BF)] * 4

    return pl.pallas_call(
        _make_block_body(bb, h, w, cin, cout, stride, proj, fin),
        out_shape=jax.ShapeDtypeStruct((n, ho, wo, cout), out_dtype),
        grid=(n // bb,),
        in_specs=in_specs,
        out_specs=pl.BlockSpec((bb, ho, wo, cout), lambda i: (i, 0, 0, 0)),
        scratch_shapes=scratch,
        compiler_params=pltpu.CompilerParams(
            dimension_semantics=("parallel",)),
        cost_estimate=pl.CostEstimate(flops=flops, transcendentals=0,
                                      bytes_accessed=bytes_accessed),
    )(*inputs)


def _make_stem_body(b, h, w, cin, cout):
    def body(x_ref, w_ref, o_ref, pad):
        _zero_border(pad, b, h + 2, w + 2, cin, jnp.float32)
        pad[:, 1:h + 1, 1:w + 1, :] = x_ref[...]
        m = b * h * w
        acc = jnp.zeros((m, cout), jnp.float32)
        for ki in range(3):
            for kj in range(3):
                win = pad[:, ki:ki + h, kj:kj + w, :]
                acc = acc + jnp.dot(win.reshape(m, cin), w_ref[ki, kj],
                                    preferred_element_type=jnp.float32)
        o_ref[...] = acc.reshape(b, h, w, cout).astype(o_ref.dtype)
    return body


def _stem(x, w, bb):
    n, h, wd, cin = x.shape
    cout = w.shape[-1]
    return pl.pallas_call(
        _make_stem_body(bb, h, wd, cin, cout),
        out_shape=jax.ShapeDtypeStruct((n, h, wd, cout), jnp.float32),
        grid=(n // bb,),
        in_specs=[pl.BlockSpec((bb, h, wd, cin), lambda i: (i, 0, 0, 0)),
                  pl.BlockSpec((3, 3, cin, cout), lambda i: (0, 0, 0, 0))],
        out_specs=pl.BlockSpec((bb, h, wd, cout), lambda i: (i, 0, 0, 0)),
        scratch_shapes=[pltpu.VMEM((bb, h + 2, wd + 2, cin), jnp.float32)],
        compiler_params=pltpu.CompilerParams(
            dimension_semantics=("parallel",)),
        cost_estimate=pl.CostEstimate(
            flops=2 * n * h * wd * 9 * cin * cout, transcendentals=0,
            bytes_accessed=(n * h * wd * (cin + cout)) * 4),
    )(x, w)


def kernel(x, stem_conv1_w, b0_bn1_scale, b0_bn1_shift, b0_conv1_w, b0_bn2_scale, b0_bn2_shift, b0_conv2_w, b0_shortcut_w, b1_bn1_scale, b1_bn1_shift, b1_conv1_w, b1_bn2_scale, b1_bn2_shift, b1_conv2_w, b2_bn1_scale, b2_bn1_shift, b2_conv1_w, b2_bn2_scale, b2_bn2_shift, b2_conv2_w, b3_bn1_scale, b3_bn1_shift, b3_conv1_w, b3_bn2_scale, b3_bn2_shift, b3_conv2_w, b4_bn1_scale, b4_bn1_shift, b4_conv1_w, b4_bn2_scale, b4_bn2_shift, b4_conv2_w, b4_shortcut_w, b5_bn1_scale, b5_bn1_shift, b5_conv1_w, b5_bn2_scale, b5_bn2_shift, b5_conv2_w, b6_bn1_scale, b6_bn1_shift, b6_conv1_w, b6_bn2_scale, b6_bn2_shift, b6_conv2_w, b7_bn1_scale, b7_bn1_shift, b7_conv1_w, b7_bn2_scale, b7_bn2_shift, b7_conv2_w, b8_bn1_scale, b8_bn1_shift, b8_conv1_w, b8_bn2_scale, b8_bn2_shift, b8_conv2_w, b8_shortcut_w, b9_bn1_scale, b9_bn1_shift, b9_conv1_w, b9_bn2_scale, b9_bn2_shift, b9_conv2_w, b10_bn1_scale, b10_bn1_shift, b10_conv1_w, b10_bn2_scale, b10_bn2_shift, b10_conv2_w, b11_bn1_scale, b11_bn1_shift, b11_conv1_w, b11_bn2_scale, b11_bn2_shift, b11_conv2_w, bn_final_scale, bn_final_shift):
    blocks = [
        (b0_bn1_scale, b0_bn1_shift, b0_conv1_w, b0_bn2_scale, b0_bn2_shift, b0_conv2_w, b0_shortcut_w, 1, 4),
        (b1_bn1_scale, b1_bn1_shift, b1_conv1_w, b1_bn2_scale, b1_bn2_shift, b1_conv2_w, None, 1, 4),
        (b2_bn1_scale, b2_bn1_shift, b2_conv1_w, b2_bn2_scale, b2_bn2_shift, b2_conv2_w, None, 1, 4),
        (b3_bn1_scale, b3_bn1_shift, b3_conv1_w, b3_bn2_scale, b3_bn2_shift, b3_conv2_w, None, 1, 4),
        (b4_bn1_scale, b4_bn1_shift, b4_conv1_w, b4_bn2_scale, b4_bn2_shift, b4_conv2_w, b4_shortcut_w, 2, 4),
        (b5_bn1_scale, b5_bn1_shift, b5_conv1_w, b5_bn2_scale, b5_bn2_shift, b5_conv2_w, None, 1, 8),
        (b6_bn1_scale, b6_bn1_shift, b6_conv1_w, b6_bn2_scale, b6_bn2_shift, b6_conv2_w, None, 1, 8),
        (b7_bn1_scale, b7_bn1_shift, b7_conv1_w, b7_bn2_scale, b7_bn2_shift, b7_conv2_w, None, 1, 8),
        (b8_bn1_scale, b8_bn1_shift, b8_conv1_w, b8_bn2_scale, b8_bn2_shift, b8_conv2_w, b8_shortcut_w, 2, 4),
        (b9_bn1_scale, b9_bn1_shift, b9_conv1_w, b9_bn2_scale, b9_bn2_shift, b9_conv2_w, None, 1, 8),
        (b10_bn1_scale, b10_bn1_shift, b10_conv1_w, b10_bn2_scale, b10_bn2_shift, b10_conv2_w, None, 1, 8),
        (b11_bn1_scale, b11_bn1_shift, b11_conv1_w, b11_bn2_scale, b11_bn2_shift, b11_conv2_w, None, 1, 8),
    ]
    h = jnp.transpose(x, (0, 2, 3, 1))
    h = _stem(h, stem_conv1_w, 8)
    last = len(blocks) - 1
    for i, (s1, t1, w1, s2, t2, w2, sw, stride, bb) in enumerate(blocks):
        post_bn = (bn_final_scale, bn_final_shift) if i == last else None
        odt = jnp.float32
        h = _res_block(h, s1, t1, w1, s2, t2, w2, sw, stride, bb, post_bn,
                       out_dtype=odt)
    return jnp.transpose(h, (0, 3, 1, 2))
